# Initial kernel scaffold; baseline (speedup 1.0000x reference)
import jax
import jax.numpy as jnp
from jax.experimental import pallas as pl


def kernel(x, edge_index, edge_attr, params):
    raise NotImplementedError("write your pallas kernel here")


# speedup vs baseline: 6.3137x; 6.3137x over previous
import jax
import jax.numpy as jnp
from jax import lax
from jax.experimental import pallas as pl
from jax.experimental.pallas import tpu as pltpu
from jax.experimental.pallas import tpu_sc as plsc

F32 = jnp.float32
I32 = jnp.int32
NS = 16
NC = 2
B = 80
NBLK = 8


def _prep(eaT, ei32):
    E = eaT.shape[1]
    Eb = 12800
    grid = E // Eb

    def body(ea_ref, ei_ref, bas_ref, srcs_ref, dsts_ref):
        t = [ea_ref[d:d + 1, :] for d in range(3)]
        for k in range(8):
            b = jnp.ones((1, Eb), F32)
            for d in range(3):
                b = b * (t[d] if (k >> d) & 1 else (1.0 - t[d]))
            bas_ref[k:k + 1, :] = b
        s0 = ei_ref[0:1, :]
        d0 = ei_ref[1:2, :]
        for l in range(4):
            srcs_ref[l:l + 1, :] = jnp.right_shift(s0, l)
            dsts_ref[l:l + 1, :] = jnp.right_shift(d0, l)

    return pl.pallas_call(
        body,
        grid=(grid,),
        in_specs=[
            pl.BlockSpec((3, Eb), lambda i: (0, i)),
            pl.BlockSpec((2, Eb), lambda i: (0, i)),
        ],
        out_specs=[
            pl.BlockSpec((8, Eb), lambda i: (0, i)),
            pl.BlockSpec((4, Eb), lambda i: (0, i)),
            pl.BlockSpec((4, Eb), lambda i: (0, i)),
        ],
        out_shape=[
            jax.ShapeDtypeStruct((8, E), F32),
            jax.ShapeDtypeStruct((4, E), I32),
            jax.ShapeDtypeStruct((4, E), I32),
        ],
    )(eaT, ei32)


def _make_sc_layer(N16, cip, lvl, do_deg, E):
    K = 4 * cip
    rpt = N16 // NS
    ept = E // NS
    nch = ept // B
    dept = E // (NS * NC)
    dnch = dept // B

    mesh = plsc.VectorSubcoreMesh(core_axis_name="c", subcore_axis_name="s")

    outs = [jax.ShapeDtypeStruct((NC, N16, K), F32)]
    scratch = [
        pltpu.VMEM_SHARED((N16, K), F32),
        pltpu.VMEM((B,), I32),
        pltpu.VMEM((B,), I32),
        pltpu.VMEM((4, B), F32),
        pltpu.VMEM((B, cip), F32),
        pltpu.VMEM((B, K), F32),
        pltpu.SemaphoreType.DMA,
    ]
    if do_deg:
        outs.append(jax.ShapeDtypeStruct((NC, N16, 16), F32))
        scratch += [
            pltpu.VMEM_SHARED((N16, 16), F32),
            pltpu.VMEM((B, 16), F32),
        ]

    def body(x_hbm, srcs_hbm, dsts_hbm, bas_hbm, zz_hbm, zd_hbm, *rest):
        if do_deg:
            (zout, dout, z_sh, src_v, dst_v, bas_v, xr_v, out_v, gsem,
             deg_sh, ones_v) = rest
        else:
            zout, z_sh, src_v, dst_v, bas_v, xr_v, out_v, gsem = rest
        c = lax.axis_index("c")
        s = lax.axis_index("s")
        r0 = s * rpt
        pltpu.sync_copy(zz_hbm.at[pl.ds(r0, rpt)], z_sh.at[pl.ds(r0, rpt)])
        if do_deg:
            pltpu.sync_copy(zd_hbm.at[pl.ds(r0, rpt)],
                            deg_sh.at[pl.ds(r0, rpt)])
        plsc.subcore_barrier()

        base = s * ept

        def chunk(j, carry):
            off = base + j * B
            pltpu.sync_copy(srcs_hbm.at[lvl, pl.ds(off, B)], src_v)
            pltpu.sync_copy(dsts_hbm.at[lvl, pl.ds(off, B)], dst_v)
            pltpu.sync_copy(bas_hbm.at[pl.ds(c * 4, 4), pl.ds(off, B)], bas_v)
            pltpu.async_copy(x_hbm.at[src_v], xr_v, gsem).wait()

            def group(g, carry2):
                e0 = g * 16
                bv = [bas_v[k, pl.ds(e0, 16)] for k in range(4)]
                for e16 in range(16):
                    e = e0 + e16
                    xrow = [xr_v[e, pl.ds(cb * 16, 16)]
                            for cb in range(cip // 16)]
                    for k in range(4):
                        bk = bv[k][e16]
                        for cb in range(cip // 16):
                            out_v[e, pl.ds(k * cip + cb * 16, 16)] = (
                                xrow[cb] * bk)
                return carry2

            lax.fori_loop(0, B // 16, group, 0)
            pltpu.sync_copy(out_v, z_sh.at[dst_v], add=True)
            return carry

        lax.fori_loop(0, nch, chunk, 0)

        if do_deg:
            onerow = jnp.where(lax.iota(I32, 16) == 0,
                               jnp.full((16,), 1.0, F32),
                               jnp.zeros((16,), F32))

            def initones(e, carry):
                ones_v[e] = onerow
                return carry

            lax.fori_loop(0, B, initones, 0)
            dbase = (c * NS + s) * dept

            def dchunk(j, carry):
                off = dbase + j * B
                pltpu.sync_copy(dsts_hbm.at[0, pl.ds(off, B)], dst_v)
                pltpu.sync_copy(ones_v, deg_sh.at[dst_v], add=True)
                return carry

            lax.fori_loop(0, dnch, dchunk, 0)

        plsc.subcore_barrier()
        pltpu.sync_copy(z_sh.at[pl.ds(r0, rpt)], zout.at[c, pl.ds(r0, rpt)])
        if do_deg:
            pltpu.sync_copy(deg_sh.at[pl.ds(r0, rpt)],
                            dout.at[c, pl.ds(r0, rpt)])

    return pl.kernel(
        body, out_type=outs, mesh=mesh, scratch_types=scratch,
        compiler_params=pltpu.CompilerParams(use_tc_tiling_on_sc=False))


def _mm(a, b):
    return lax.dot_general(a, b, (((1,), (0,)), ((), ())),
                           precision=lax.Precision.DEFAULT,
                           preferred_element_type=F32)


def _make_dense(N, N16, K, cip, co, first):
    Nb = N16 // NBLK

    def body(z_ref, x_ref, wlo_ref, whi_ref, root_ref, bias_ref, deg_ref,
             h_ref, stats_ref, degc_ref=None):
        i = pl.program_id(0)
        if first:
            dv = deg_ref[0] + deg_ref[1]
            degcol = dv[:, 0:1]
            degc_ref[...] = degcol
        else:
            degcol = deg_ref[...]
        agg = _mm(z_ref[0], wlo_ref[...]) + _mm(z_ref[1], whi_ref[...])
        agg = agg / jnp.clip(degcol, 1.0, None)
        h = agg + _mm(x_ref[...], root_ref[...]) + bias_ref[...]
        h = jnp.where(h > 0, h, jnp.exp(jnp.minimum(h, 0.0)) - 1.0)
        h_ref[...] = h
        row = i * Nb + lax.broadcasted_iota(I32, (Nb, 1), 0)
        hm = jnp.where(row < N, h, 0.0)
        st = jnp.concatenate([jnp.sum(hm, axis=0, keepdims=True),
                              jnp.sum(hm * hm, axis=0, keepdims=True)], 0)

        @pl.when(i == 0)
        def _():
            stats_ref[...] = st

        @pl.when(i > 0)
        def _():
            stats_ref[...] = stats_ref[...] + st

    deg_spec = (pl.BlockSpec((2, Nb, 16), lambda i: (0, i, 0)) if first
                else pl.BlockSpec((Nb, 1), lambda i: (i, 0)))
    outs = [jax.ShapeDtypeStruct((N16, co), F32),
            jax.ShapeDtypeStruct((2, co), F32)]
    out_specs = [pl.BlockSpec((Nb, co), lambda i: (i, 0)),
                 pl.BlockSpec((2, co), lambda i: (0, 0))]
    if first:
        outs.append(jax.ShapeDtypeStruct((N16, 1), F32))
        out_specs.append(pl.BlockSpec((Nb, 1), lambda i: (i, 0)))
    return pl.pallas_call(
        body,
        grid=(NBLK,),
        in_specs=[
            pl.BlockSpec((2, Nb, K), lambda i: (0, i, 0)),
            pl.BlockSpec((Nb, cip), lambda i: (i, 0)),
            pl.BlockSpec((K, co), lambda i: (0, 0)),
            pl.BlockSpec((K, co), lambda i: (0, 0)),
            pl.BlockSpec((cip, co), lambda i: (0, 0)),
            pl.BlockSpec((1, co), lambda i: (0, 0)),
            deg_spec,
        ],
        out_specs=out_specs,
        out_shape=outs,
    )


def _bn_coeffs(stats_ref, g_ref, be_ref, N):
    mu = stats_ref[0:1, :] / float(N)
    var = stats_ref[1:2, :] / float(N) - mu * mu
    scale = g_ref[...] * lax.rsqrt(var + 1e-5)
    shift = be_ref[...] - mu * scale
    return scale, shift


def _make_bn(N, N16, co):
    def body(h_ref, stats_ref, g_ref, be_ref, y_ref):
        scale, shift = _bn_coeffs(stats_ref, g_ref, be_ref, N)
        y_ref[...] = h_ref[...] * scale + shift

    return pl.pallas_call(
        body, out_shape=[jax.ShapeDtypeStruct((N16, co), F32)])


def _make_bn_pool(N, N16, co):
    N2 = N16 // 2

    def body(hr_ref, stats_ref, g_ref, be_ref, degr_ref, xp_ref, degp_ref):
        scale, shift = _bn_coeffs(stats_ref, g_ref, be_ref, N)
        a = hr_ref[:, :co] * scale + shift
        b = hr_ref[:, co:] * scale + shift
        xp_ref[...] = jnp.maximum(a, b)
        degp_ref[...] = degr_ref[:, 0:1] + degr_ref[:, 1:2]

    return pl.pallas_call(
        body,
        out_shape=[jax.ShapeDtypeStruct((N2, co), F32),
                   jax.ShapeDtypeStruct((N2, 1), F32)],
    )


_CFG = [
    dict(N=10000, N16=10240, cip=16, co=32, lvl=0, pool=False),
    dict(N=10000, N16=10240, cip=32, co=64, lvl=0, pool=True),
    dict(N=5000, N16=5120, cip=64, co=64, lvl=1, pool=True),
    dict(N=2500, N16=2560, cip=64, co=64, lvl=2, pool=True),
    dict(N=1250, N16=1280, cip=64, co=128, lvl=3, pool=False),
    dict(N=1250, N16=1280, cip=128, co=256, lvl=3, pool=False),
]


def kernel(x, edge_index, edge_attr, params):
    E = edge_attr.shape[0]
    ei32 = edge_index.astype(I32)
    eaT = edge_attr.T

    basT, srcs, dsts = _prep(eaT, ei32)

    xcur = jnp.pad(x, ((0, 10240 - x.shape[0]), (0, 16 - x.shape[1])))
    degcol = None
    for i, cfg in enumerate(_CFG, start=1):
        N, N16, cip, co, lvl = (cfg['N'], cfg['N16'], cfg['cip'], cfg['co'],
                                cfg['lvl'])
        K = 4 * cip
        first = (i == 1)

        W = params['W%d' % i]
        ci = W.shape[1]
        Wp = jnp.pad(W, ((0, 0), (0, cip - ci), (0, 0)))
        wlo = Wp[0:4].reshape(K, co)
        whi = Wp[4:8].reshape(K, co)
        rootp = jnp.pad(params['r%d' % i], ((0, cip - ci), (0, 0)))

        sc = _make_sc_layer(N16, cip, lvl, first, E)
        zz = jnp.zeros((N16, K), F32)
        zd = jnp.zeros((N16, 16), F32)
        if first:
            Z, D = sc(xcur, srcs, dsts, basT, zz, zd)
            degin = D
        else:
            (Z,) = sc(xcur, srcs, dsts, basT, zz, zd)
            degin = degcol

        dense = _make_dense(N, N16, K, cip, co, first)
        if first:
            h, stats, degcol = dense(Z, xcur, wlo, whi, rootp,
                                     params['b%d' % i].reshape(1, co), degin)
        else:
            h, stats = dense(Z, xcur, wlo, whi, rootp,
                             params['b%d' % i].reshape(1, co), degin)

        g = params['g%d' % i].reshape(1, co)
        be = params['be%d' % i].reshape(1, co)
        if cfg['pool']:
            hr = h.reshape(N16 // 2, 2 * co)
            degr = degcol.reshape(N16 // 2, 2)
            xcur, degcol = _make_bn_pool(N, N16, co)(hr, stats, g, be, degr)
        else:
            (xcur,) = _make_bn(N, N16, co)(h, stats, g, be)
    return xcur[:1250]


# speedup vs baseline: 11.9441x; 1.8918x over previous
import jax
import jax.numpy as jnp
from jax import lax
from jax.experimental import pallas as pl
from jax.experimental.pallas import tpu as pltpu
from jax.experimental.pallas import tpu_sc as plsc

F32 = jnp.float32
I32 = jnp.int32
NS = 16
NC = 2
B = 80
NBLK = 8


def _prep(eaT, ei32):
    E = eaT.shape[1]
    Eb = 12800
    grid = E // Eb

    def body(ea_ref, ei_ref, bas_ref, sd_ref):
        t = [ea_ref[d:d + 1, :] for d in range(3)]
        for k in range(8):
            b = jnp.ones((1, Eb), F32)
            for d in range(3):
                b = b * (t[d] if (k >> d) & 1 else (1.0 - t[d]))
            bas_ref[k:k + 1, :] = b
        s0 = ei_ref[0:1, :]
        d0 = ei_ref[1:2, :]
        for l in range(4):
            sd_ref[l, 0:1, :] = jnp.right_shift(s0, l)
            sd_ref[l, 1:2, :] = jnp.right_shift(d0, l)

    return pl.pallas_call(
        body,
        grid=(grid,),
        in_specs=[
            pl.BlockSpec((3, Eb), lambda i: (0, i)),
            pl.BlockSpec((2, Eb), lambda i: (0, i)),
        ],
        out_specs=[
            pl.BlockSpec((8, Eb), lambda i: (0, i)),
            pl.BlockSpec((4, 2, Eb), lambda i: (0, 0, i)),
        ],
        out_shape=[
            jax.ShapeDtypeStruct((8, E), F32),
            jax.ShapeDtypeStruct((4, 2, E), I32),
        ],
    )(eaT, ei32)


def _make_sc_layer(N16, cip, lvl, do_deg, E):
    K = 4 * cip
    rpt = N16 // NS
    ept = E // NS
    nch = ept // B
    ng = nch // 2
    dept = E // (NS * NC)
    dnch = dept // B

    mesh = plsc.VectorSubcoreMesh(core_axis_name="c", subcore_axis_name="s")

    words2 = N16 * K + NS * (2 * B * K + 2 * B * cip + 16 * B)
    if do_deg:
        words2 += N16 * 16 + NS * B * 16
    OB = 2 if words2 < 1_950_000 else 1

    outs = [jax.ShapeDtypeStruct((NC, N16, K), F32)]
    scratch = [
        pltpu.VMEM_SHARED((N16, K), F32),
        pltpu.VMEM((2, 2, B), I32),
        pltpu.VMEM((OB, B), I32),
        pltpu.VMEM((2, 4, B), F32),
        pltpu.VMEM((2, B, cip), F32),
        pltpu.VMEM((OB, B, K), F32),
    ] + [pltpu.SemaphoreType.DMA] * 8
    if do_deg:
        outs.append(jax.ShapeDtypeStruct((NC, N16, 16), F32))
        scratch += [
            pltpu.VMEM_SHARED((N16, 16), F32),
            pltpu.VMEM((B, 16), F32),
        ]

    def body(x_hbm, sd_hbm, bas_hbm, zz_hbm, zd_hbm, *rest):
        if do_deg:
            (zout, dout, z_sh, sdx_v, sidx_v, bas_v, xr_v, out_v,
             i0, i1, b0, b1, g0, g1, s0, s1, deg_sh, ones_v) = rest
        else:
            (zout, z_sh, sdx_v, sidx_v, bas_v, xr_v, out_v,
             i0, i1, b0, b1, g0, g1, s0, s1) = rest
        isem = (i0, i1)
        bsem = (b0, b1)
        gsem = (g0, g1)
        ssem = (s0, s1)
        c = lax.axis_index("c")
        s = lax.axis_index("s")
        r0 = s * rpt
        pltpu.sync_copy(zz_hbm.at[pl.ds(r0, rpt)], z_sh.at[pl.ds(r0, rpt)])
        if do_deg:
            pltpu.sync_copy(zd_hbm.at[pl.ds(r0, rpt)],
                            deg_sh.at[pl.ds(r0, rpt)])
        plsc.subcore_barrier()

        base = s * ept

        def start_idx(j, p):
            off = base + j * B
            pltpu.async_copy(sd_hbm.at[lvl, :, pl.ds(off, B)],
                             sdx_v.at[p], isem[p])
            pltpu.async_copy(bas_hbm.at[pl.ds(c * 4, 4), pl.ds(off, B)],
                             bas_v.at[p], bsem[p])

        def start_gather(p):
            pltpu.async_copy(x_hbm.at[sdx_v.at[p, 0]], xr_v.at[p], gsem[p])

        def compute(p, po):
            def group(g, carry2):
                e0 = g * 16
                bv = [bas_v[p, k, pl.ds(e0, 16)] for k in range(4)]
                for e16 in range(16):
                    e = e0 + e16
                    xrow = [xr_v[p, e, pl.ds(cb * 16, 16)]
                            for cb in range(cip // 16)]
                    for k in range(4):
                        bk = bv[k][e16]
                        for cb in range(cip // 16):
                            out_v[po, e, pl.ds(k * cip + cb * 16, 16)] = (
                                xrow[cb] * bk)
                return carry2

            lax.fori_loop(0, B // 16, group, 0)

        start_idx(0, 0)
        pltpu.make_async_copy(sd_hbm.at[lvl, :, pl.ds(base, B)],
                              sdx_v.at[0], isem[0]).wait()
        start_gather(0)

        def outer(g, carry):
            for p in (0, 1):
                q = 1 - p
                j = 2 * g + p
                if p == 0:
                    start_idx(j + 1, q)
                else:
                    @pl.when(g < ng - 1)
                    def _():
                        start_idx(j + 1, q)
                po = p % OB
                pltpu.make_async_copy(x_hbm.at[sdx_v.at[p, 0]],
                                      xr_v.at[p], gsem[p]).wait()

                def wait_scatter():
                    pltpu.make_async_copy(out_v.at[po],
                                          z_sh.at[sidx_v.at[po]],
                                          ssem[po]).wait()

                if OB == 2 or p == 0:
                    @pl.when(g >= 1)
                    def _():
                        wait_scatter()
                else:
                    wait_scatter()
                pltpu.make_async_copy(
                    bas_hbm.at[pl.ds(c * 4, 4), pl.ds(0, B)],
                    bas_v.at[p], bsem[p]).wait()
                compute(p, po)
                for i in range(B // 16):
                    sidx_v[po, pl.ds(i * 16, 16)] = (
                        sdx_v[p, 1, pl.ds(i * 16, 16)])
                pltpu.async_copy(out_v.at[po], z_sh.at[sidx_v.at[po]],
                                 ssem[po], add=True)
                if p == 0:
                    pltpu.make_async_copy(
                        sd_hbm.at[lvl, :, pl.ds(base, B)],
                        sdx_v.at[q], isem[q]).wait()
                    start_gather(q)
                else:
                    @pl.when(g < ng - 1)
                    def _():
                        pltpu.make_async_copy(
                            sd_hbm.at[lvl, :, pl.ds(base, B)],
                            sdx_v.at[q], isem[q]).wait()
                        start_gather(q)
            return carry

        lax.fori_loop(0, ng, outer, 0)
        for po in range(OB):
            pltpu.make_async_copy(out_v.at[po], z_sh.at[sidx_v.at[po]],
                                  ssem[po]).wait()

        if do_deg:
            onerow = jnp.where(lax.iota(I32, 16) == 0,
                               jnp.full((16,), 1.0, F32),
                               jnp.zeros((16,), F32))

            def initones(e, carry):
                ones_v[e] = onerow
                return carry

            lax.fori_loop(0, B, initones, 0)
            dbase = (c * NS + s) * dept

            def dchunk(j, carry):
                off = dbase + j * B
                pltpu.sync_copy(sd_hbm.at[0, 1, pl.ds(off, B)],
                                sidx_v.at[0])
                pltpu.sync_copy(ones_v, deg_sh.at[sidx_v.at[0]], add=True)
                return carry

            lax.fori_loop(0, dnch, dchunk, 0)

        plsc.subcore_barrier()
        pltpu.sync_copy(z_sh.at[pl.ds(r0, rpt)], zout.at[c, pl.ds(r0, rpt)])
        if do_deg:
            pltpu.sync_copy(deg_sh.at[pl.ds(r0, rpt)],
                            dout.at[c, pl.ds(r0, rpt)])

    return pl.kernel(
        body, out_type=outs, mesh=mesh, scratch_types=scratch,
        compiler_params=pltpu.CompilerParams(use_tc_tiling_on_sc=False))


def _mm(a, b):
    return lax.dot_general(a, b, (((1,), (0,)), ((), ())),
                           precision=lax.Precision.DEFAULT,
                           preferred_element_type=F32)


def _make_dense(N, N16, K, cip, co, first):
    Nb = N16 // NBLK

    def body(z_ref, x_ref, wlo_ref, whi_ref, root_ref, bias_ref, deg_ref,
             h_ref, stats_ref, degc_ref=None):
        i = pl.program_id(0)
        if first:
            dv = deg_ref[0] + deg_ref[1]
            degcol = dv[:, 0:1]
            degc_ref[...] = degcol
        else:
            degcol = deg_ref[...]
        agg = _mm(z_ref[0], wlo_ref[...]) + _mm(z_ref[1], whi_ref[...])
        agg = agg / jnp.clip(degcol, 1.0, None)
        h = agg + _mm(x_ref[...], root_ref[...]) + bias_ref[...]
        h = jnp.where(h > 0, h, jnp.exp(jnp.minimum(h, 0.0)) - 1.0)
        h_ref[...] = h
        row = i * Nb + lax.broadcasted_iota(I32, (Nb, 1), 0)
        hm = jnp.where(row < N, h, 0.0)
        st = jnp.concatenate([jnp.sum(hm, axis=0, keepdims=True),
                              jnp.sum(hm * hm, axis=0, keepdims=True)], 0)

        @pl.when(i == 0)
        def _():
            stats_ref[...] = st

        @pl.when(i > 0)
        def _():
            stats_ref[...] = stats_ref[...] + st

    deg_spec = (pl.BlockSpec((2, Nb, 16), lambda i: (0, i, 0)) if first
                else pl.BlockSpec((Nb, 1), lambda i: (i, 0)))
    outs = [jax.ShapeDtypeStruct((N16, co), F32),
            jax.ShapeDtypeStruct((2, co), F32)]
    out_specs = [pl.BlockSpec((Nb, co), lambda i: (i, 0)),
                 pl.BlockSpec((2, co), lambda i: (0, 0))]
    if first:
        outs.append(jax.ShapeDtypeStruct((N16, 1), F32))
        out_specs.append(pl.BlockSpec((Nb, 1), lambda i: (i, 0)))
    return pl.pallas_call(
        body,
        grid=(NBLK,),
        in_specs=[
            pl.BlockSpec((2, Nb, K), lambda i: (0, i, 0)),
            pl.BlockSpec((Nb, cip), lambda i: (i, 0)),
            pl.BlockSpec((K, co), lambda i: (0, 0)),
            pl.BlockSpec((K, co), lambda i: (0, 0)),
            pl.BlockSpec((cip, co), lambda i: (0, 0)),
            pl.BlockSpec((1, co), lambda i: (0, 0)),
            deg_spec,
        ],
        out_specs=out_specs,
        out_shape=outs,
    )


def _bn_coeffs(stats_ref, g_ref, be_ref, N):
    mu = stats_ref[0:1, :] / float(N)
    var = stats_ref[1:2, :] / float(N) - mu * mu
    scale = g_ref[...] * lax.rsqrt(var + 1e-5)
    shift = be_ref[...] - mu * scale
    return scale, shift


def _make_bn(N, N16, co):
    def body(h_ref, stats_ref, g_ref, be_ref, y_ref):
        scale, shift = _bn_coeffs(stats_ref, g_ref, be_ref, N)
        y_ref[...] = h_ref[...] * scale + shift

    return pl.pallas_call(
        body, out_shape=[jax.ShapeDtypeStruct((N16, co), F32)])


def _make_bn_pool(N, N16, co):
    N2 = N16 // 2

    def body(hr_ref, stats_ref, g_ref, be_ref, degr_ref, xp_ref, degp_ref):
        scale, shift = _bn_coeffs(stats_ref, g_ref, be_ref, N)
        a = hr_ref[:, :co] * scale + shift
        b = hr_ref[:, co:] * scale + shift
        xp_ref[...] = jnp.maximum(a, b)
        degp_ref[...] = degr_ref[:, 0:1] + degr_ref[:, 1:2]

    return pl.pallas_call(
        body,
        out_shape=[jax.ShapeDtypeStruct((N2, co), F32),
                   jax.ShapeDtypeStruct((N2, 1), F32)],
    )


_CFG = [
    dict(N=10000, N16=10240, cip=16, co=32, lvl=0, pool=False),
    dict(N=10000, N16=10240, cip=32, co=64, lvl=0, pool=True),
    dict(N=5000, N16=5120, cip=64, co=64, lvl=1, pool=True),
    dict(N=2500, N16=2560, cip=64, co=64, lvl=2, pool=True),
    dict(N=1250, N16=1280, cip=64, co=128, lvl=3, pool=False),
    dict(N=1250, N16=1280, cip=128, co=256, lvl=3, pool=False),
]


def kernel(x, edge_index, edge_attr, params):
    E = edge_attr.shape[0]
    ei32 = edge_index.astype(I32)
    eaT = edge_attr.T

    basT, sd = _prep(eaT, ei32)

    xcur = jnp.pad(x, ((0, 10240 - x.shape[0]), (0, 16 - x.shape[1])))
    degcol = None
    for i, cfg in enumerate(_CFG, start=1):
        N, N16, cip, co, lvl = (cfg['N'], cfg['N16'], cfg['cip'], cfg['co'],
                                cfg['lvl'])
        K = 4 * cip
        first = (i == 1)

        W = params['W%d' % i]
        ci = W.shape[1]
        Wp = jnp.pad(W, ((0, 0), (0, cip - ci), (0, 0)))
        wlo = Wp[0:4].reshape(K, co)
        whi = Wp[4:8].reshape(K, co)
        rootp = jnp.pad(params['r%d' % i], ((0, cip - ci), (0, 0)))

        sc = _make_sc_layer(N16, cip, lvl, first, E)
        zz = jnp.zeros((N16, K), F32)
        zd = jnp.zeros((N16, 16), F32)
        if first:
            Z, D = sc(xcur, sd, basT, zz, zd)
            degin = D
        else:
            (Z,) = sc(xcur, sd, basT, zz, zd)
            degin = degcol

        dense = _make_dense(N, N16, K, cip, co, first)
        if first:
            h, stats, degcol = dense(Z, xcur, wlo, whi, rootp,
                                     params['b%d' % i].reshape(1, co), degin)
        else:
            h, stats = dense(Z, xcur, wlo, whi, rootp,
                             params['b%d' % i].reshape(1, co), degin)

        g = params['g%d' % i].reshape(1, co)
        be = params['be%d' % i].reshape(1, co)
        if cfg['pool']:
            hr = h.reshape(N16 // 2, 2 * co)
            degr = degcol.reshape(N16 // 2, 2)
            xcur, degcol = _make_bn_pool(N, N16, co)(hr, stats, g, be, degr)
        else:
            (xcur,) = _make_bn(N, N16, co)(h, stats, g, be)
    return xcur[:1250]
